# Initial kernel scaffold; baseline (speedup 1.0000x reference)
import jax
import jax.numpy as jnp
from jax.experimental import pallas as pl


def kernel(feature, s_adj, t_adj, poi_adj, sW, s_al, s_ar, s_b, tW, t_al, t_ar, t_b, pW, p_al, p_ar, p_b, qW, qb, kW, kb, mvW, mvb):
    raise NotImplementedError("write your pallas kernel here")


# speedup vs baseline: 1.0816x; 1.0816x over previous
import functools

import jax
import jax.numpy as jnp
from jax import lax
from jax.experimental import pallas as pl
from jax.experimental.pallas import tpu as pltpu

N = 1024
DIN = 256
H = 8
DH = 32
HDH = H * DH
NEG_SLOPE = 0.2
ALPHA = 0.8
BETA = 0.5

BU = 256
NU = N // BU


def _gat_kernel(feat_ref, adj_ref, W_ref, alm_ref, arm_ref, b_ref, out_ref,
                h_s, el_s, erT_s, MT_s, num_s, den_s):
    ui = pl.program_id(1)

    @pl.when(ui == 0)
    def _setup():
        h = jnp.dot(feat_ref[...], W_ref[0], preferred_element_type=jnp.float32)
        h_s[...] = h
        el_s[...] = jnp.dot(h, alm_ref[0], preferred_element_type=jnp.float32)
        dn = (((0,), (1,)), ((), ()))
        erT = lax.dot_general(arm_ref[0], h, dn, preferred_element_type=jnp.float32)
        erT_s[...] = erT
        elT = lax.dot_general(alm_ref[0], h, dn, preferred_element_type=jnp.float32)
        m = jnp.max(elT, axis=1, keepdims=True) + erT
        MT_s[...] = jnp.where(m > 0, m, NEG_SLOPE * m)
        num_s[...] = jnp.zeros_like(num_s)
        den_s[...] = jnp.zeros_like(den_s)

    adj = adj_ref[0]
    rows = ui * BU + lax.broadcasted_iota(jnp.int32, (BU, N), 0)
    cols = lax.broadcasted_iota(jnp.int32, (BU, N), 1)
    mask = (adj > 0.0) | (rows == cols)
    ones_col = jnp.ones((BU, 1), dtype=jnp.float32)
    dn0 = (((0,), (0,)), ((), ()))

    for hh in range(H):
        el_col = el_s[pl.ds(ui * BU, BU), hh:hh + 1]
        er_row = erT_s[hh:hh + 1, :]
        e = el_col + er_row
        e = jnp.where(e > 0, e, NEG_SLOPE * e)
        p = jnp.where(mask, jnp.exp(e - MT_s[hh:hh + 1, :]), 0.0)
        hsl = h_s[pl.ds(ui * BU, BU), hh * DH:(hh + 1) * DH]
        num_s[:, hh * DH:(hh + 1) * DH] += lax.dot_general(
            p, hsl, dn0, preferred_element_type=jnp.float32)
        den_s[:, hh:hh + 1] += lax.dot_general(
            p, ones_col, dn0, preferred_element_type=jnp.float32)

    @pl.when(ui == NU - 1)
    def _finish():
        for hh in range(H):
            sl = slice(hh * DH, (hh + 1) * DH)
            o = num_s[:, sl] / den_s[:, hh:hh + 1] + b_ref[0, 0, sl]
            out_ref[0, :, sl] = jnp.maximum(o, 0.0)


def _combine_kernel(single_ref, qW_ref, qb_ref, kW_ref, kb_ref, mvW_ref,
                    mvb_ref, mv_ref, res_ref):
    d_k = jnp.sqrt(jnp.float32(DH * N))
    logits = []
    gs = []
    views = []
    for i in range(3):
        si = single_ref[i]
        views.append(si)
        Qi = jnp.dot(si, qW_ref[...], preferred_element_type=jnp.float32) + qb_ref[0]
        Ki = jnp.dot(si, kW_ref[...], preferred_element_type=jnp.float32) + kb_ref[0]
        logits.append(jnp.sum(Qi * Ki) / d_k)
        gs.append(jnp.sum(si * mvW_ref[...]))
    m = jnp.maximum(jnp.maximum(logits[0], logits[1]), logits[2])
    ex = [jnp.exp(l - m) for l in logits]
    tot = ex[0] + ex[1] + ex[2]
    mvb = mvb_ref[0, 0]
    c = [ALPHA * (e / tot) + (1.0 - ALPHA) for e in ex]
    omega = [jax.nn.sigmoid(c[i] * gs[i] + mvb) for i in range(3)]
    mv = (omega[0] * c[0] * views[0] + omega[1] * c[1] * views[1]
          + omega[2] * c[2] * views[2])
    mv_ref[...] = mv
    for i in range(3):
        res_ref[i] = BETA * c[i] * views[i] + (1.0 - BETA) * mv


def _block_diag_attn(a):
    out = jnp.zeros((H, DH, H), dtype=a.dtype)
    out = out.at[jnp.arange(H), :, jnp.arange(H)].set(a)
    return out.reshape(HDH, H)


@jax.jit
def kernel(feature, s_adj, t_adj, poi_adj,
           sW, s_al, s_ar, s_b,
           tW, t_al, t_ar, t_b,
           pW, p_al, p_ar, p_b,
           qW, qb, kW, kb, mvW, mvb):
    adj_all = jnp.stack([s_adj, t_adj, poi_adj])
    W_all = jnp.stack([sW, tW, pW])
    alm_all = jnp.stack([_block_diag_attn(a) for a in (s_al, t_al, p_al)])
    arm_all = jnp.stack([_block_diag_attn(a) for a in (s_ar, t_ar, p_ar)])
    b_all = jnp.stack([s_b, t_b, p_b]).reshape(3, 1, HDH)

    single = pl.pallas_call(
        _gat_kernel,
        grid=(3, NU),
        in_specs=[
            pl.BlockSpec((N, DIN), lambda vi, ui: (0, 0)),
            pl.BlockSpec((1, BU, N), lambda vi, ui: (vi, ui, 0)),
            pl.BlockSpec((1, DIN, HDH), lambda vi, ui: (vi, 0, 0)),
            pl.BlockSpec((1, HDH, H), lambda vi, ui: (vi, 0, 0)),
            pl.BlockSpec((1, HDH, H), lambda vi, ui: (vi, 0, 0)),
            pl.BlockSpec((1, 1, HDH), lambda vi, ui: (vi, 0, 0)),
        ],
        out_specs=pl.BlockSpec((1, N, HDH), lambda vi, ui: (vi, 0, 0)),
        out_shape=jax.ShapeDtypeStruct((3, N, HDH), jnp.float32),
        scratch_shapes=[
            pltpu.VMEM((N, HDH), jnp.float32),
            pltpu.VMEM((N, H), jnp.float32),
            pltpu.VMEM((H, N), jnp.float32),
            pltpu.VMEM((H, N), jnp.float32),
            pltpu.VMEM((N, HDH), jnp.float32),
            pltpu.VMEM((N, H), jnp.float32),
        ],
    )(feature, adj_all, W_all, alm_all, arm_all, b_all)

    mv_out, result = pl.pallas_call(
        _combine_kernel,
        grid=(1,),
        in_specs=[
            pl.BlockSpec((3, N, HDH), lambda i: (0, 0, 0)),
            pl.BlockSpec((HDH, DH), lambda i: (0, 0)),
            pl.BlockSpec((1, DH), lambda i: (0, 0)),
            pl.BlockSpec((HDH, DH), lambda i: (0, 0)),
            pl.BlockSpec((1, DH), lambda i: (0, 0)),
            pl.BlockSpec((N, HDH), lambda i: (0, 0)),
            pl.BlockSpec((1, 1), lambda i: (0, 0)),
        ],
        out_specs=[
            pl.BlockSpec((N, HDH), lambda i: (0, 0)),
            pl.BlockSpec((3, N, HDH), lambda i: (0, 0, 0)),
        ],
        out_shape=[
            jax.ShapeDtypeStruct((N, HDH), jnp.float32),
            jax.ShapeDtypeStruct((3, N, HDH), jnp.float32),
        ],
    )(single, qW, qb.reshape(1, DH), kW, kb.reshape(1, DH),
      mvW.reshape(N, HDH), mvb.reshape(1, 1))

    return (mv_out, result)


# speedup vs baseline: 1.1385x; 1.0525x over previous
import jax
import jax.numpy as jnp
from jax import lax
from jax.experimental import pallas as pl
from jax.experimental.pallas import tpu as pltpu

N = 1024
DIN = 256
H = 8
DH = 32
HDH = H * DH
NEG_SLOPE = 0.2
ALPHA = 0.8
BETA = 0.5

BU = 256
NU = N // BU


def _fused_kernel(feat_ref,
                  adj0_ref, adj1_ref, adj2_ref,
                  W0_ref, alm0_ref, arm0_ref, b0_ref,
                  W1_ref, alm1_ref, arm1_ref, b1_ref,
                  W2_ref, alm2_ref, arm2_ref, b2_ref,
                  qW_ref, qb_ref, kW_ref, kb_ref, mvW_ref, mvb_ref,
                  mv_ref, res_ref,
                  h_s, el_s, erT_s, MT_s, num_s, den_s):
    ui = pl.program_id(0)
    adj_refs = (adj0_ref, adj1_ref, adj2_ref)
    W_refs = (W0_ref, W1_ref, W2_ref)
    alm_refs = (alm0_ref, alm1_ref, alm2_ref)
    arm_refs = (arm0_ref, arm1_ref, arm2_ref)
    b_refs = (b0_ref, b1_ref, b2_ref)

    @pl.when(ui == 0)
    def _setup():
        feat = feat_ref[...]
        for v in range(3):
            h = jnp.dot(feat, W_refs[v][...], preferred_element_type=jnp.float32)
            h_s[v] = h
            el_s[v] = jnp.dot(h, alm_refs[v][...], preferred_element_type=jnp.float32)
            dn = (((0,), (1,)), ((), ()))
            erT = lax.dot_general(arm_refs[v][...], h, dn,
                                  preferred_element_type=jnp.float32)
            erT_s[v] = erT
            elT = lax.dot_general(alm_refs[v][...], h, dn,
                                  preferred_element_type=jnp.float32)
            m = jnp.max(elT, axis=1, keepdims=True) + erT
            MT_s[v] = jnp.where(m > 0, m, NEG_SLOPE * m)
        num_s[...] = jnp.zeros_like(num_s)
        den_s[...] = jnp.zeros_like(den_s)

    rows = ui * BU + lax.broadcasted_iota(jnp.int32, (BU, N), 0)
    cols = lax.broadcasted_iota(jnp.int32, (BU, N), 1)
    diag = rows == cols
    ones_col = jnp.ones((BU, 1), dtype=jnp.float32)
    dn0 = (((0,), (0,)), ((), ()))

    for v in range(3):
        mask = (adj_refs[v][...] > 0.0) | diag
        for hh in range(H):
            el_col = el_s[v, pl.ds(ui * BU, BU), hh:hh + 1]
            er_row = erT_s[v, hh:hh + 1, :]
            e = el_col + er_row
            e = jnp.where(e > 0, e, NEG_SLOPE * e)
            p = jnp.where(mask, jnp.exp(e - MT_s[v, hh:hh + 1, :]), 0.0)
            hsl = h_s[v, pl.ds(ui * BU, BU), hh * DH:(hh + 1) * DH]
            num_s[v, :, hh * DH:(hh + 1) * DH] += lax.dot_general(
                p, hsl, dn0, preferred_element_type=jnp.float32)
            den_s[v, :, hh:hh + 1] += lax.dot_general(
                p, ones_col, dn0, preferred_element_type=jnp.float32)

    @pl.when(ui == NU - 1)
    def _finish():
        views = []
        for v in range(3):
            cols_out = []
            for hh in range(H):
                sl = slice(hh * DH, (hh + 1) * DH)
                o = (num_s[v, :, sl] / den_s[v, :, hh:hh + 1]
                     + b_refs[v][0, sl])
                cols_out.append(jnp.maximum(o, 0.0))
            views.append(jnp.concatenate(cols_out, axis=1))

        d_k = jnp.sqrt(jnp.float32(DH * N))
        qW = qW_ref[...]
        kW = kW_ref[...]
        mvW = mvW_ref[...]
        logits = []
        gs = []
        for v in range(3):
            Qv = jnp.dot(views[v], qW, preferred_element_type=jnp.float32) + qb_ref[0]
            Kv = jnp.dot(views[v], kW, preferred_element_type=jnp.float32) + kb_ref[0]
            logits.append(jnp.sum(Qv * Kv) / d_k)
            gs.append(jnp.sum(views[v] * mvW))
        m = jnp.maximum(jnp.maximum(logits[0], logits[1]), logits[2])
        ex = [jnp.exp(l - m) for l in logits]
        tot = ex[0] + ex[1] + ex[2]
        mvb = mvb_ref[0, 0]
        c = [ALPHA * (e / tot) + (1.0 - ALPHA) for e in ex]
        omega = [jax.nn.sigmoid(c[v] * gs[v] + mvb) for v in range(3)]
        mv = (omega[0] * c[0] * views[0] + omega[1] * c[1] * views[1]
              + omega[2] * c[2] * views[2])
        mv_ref[...] = mv
        for v in range(3):
            res_ref[v] = BETA * c[v] * views[v] + (1.0 - BETA) * mv


def _block_diag_attn(a):
    out = jnp.zeros((H, DH, H), dtype=a.dtype)
    out = out.at[jnp.arange(H), :, jnp.arange(H)].set(a)
    return out.reshape(HDH, H)


@jax.jit
def kernel(feature, s_adj, t_adj, poi_adj,
           sW, s_al, s_ar, s_b,
           tW, t_al, t_ar, t_b,
           pW, p_al, p_ar, p_b,
           qW, qb, kW, kb, mvW, mvb):
    full = lambda *shape: pl.BlockSpec(shape, lambda ui: (0,) * len(shape))
    per_view_specs = []
    for _ in range(3):
        per_view_specs += [
            full(DIN, HDH),
            full(HDH, H),
            full(HDH, H),
            full(1, HDH),
        ]

    mv_out, result = pl.pallas_call(
        _fused_kernel,
        grid=(NU,),
        in_specs=[
            full(N, DIN),
            pl.BlockSpec((BU, N), lambda ui: (ui, 0)),
            pl.BlockSpec((BU, N), lambda ui: (ui, 0)),
            pl.BlockSpec((BU, N), lambda ui: (ui, 0)),
            *per_view_specs,
            full(HDH, DH),
            full(1, DH),
            full(HDH, DH),
            full(1, DH),
            full(N, HDH),
            full(1, 1),
        ],
        out_specs=[
            full(N, HDH),
            pl.BlockSpec((3, N, HDH), lambda ui: (0, 0, 0)),
        ],
        out_shape=[
            jax.ShapeDtypeStruct((N, HDH), jnp.float32),
            jax.ShapeDtypeStruct((3, N, HDH), jnp.float32),
        ],
        scratch_shapes=[
            pltpu.VMEM((3, N, HDH), jnp.float32),
            pltpu.VMEM((3, N, H), jnp.float32),
            pltpu.VMEM((3, H, N), jnp.float32),
            pltpu.VMEM((3, H, N), jnp.float32),
            pltpu.VMEM((3, N, HDH), jnp.float32),
            pltpu.VMEM((3, N, H), jnp.float32),
        ],
    )(feature, s_adj, t_adj, poi_adj,
      sW, _block_diag_attn(s_al), _block_diag_attn(s_ar), s_b.reshape(1, HDH),
      tW, _block_diag_attn(t_al), _block_diag_attn(t_ar), t_b.reshape(1, HDH),
      pW, _block_diag_attn(p_al), _block_diag_attn(p_ar), p_b.reshape(1, HDH),
      qW, qb.reshape(1, DH), kW, kb.reshape(1, DH),
      mvW.reshape(N, HDH), mvb.reshape(1, 1))

    return (mv_out, result)


# speedup vs baseline: 1.2669x; 1.1128x over previous
import jax
import jax.numpy as jnp
from jax import lax
from jax.experimental import pallas as pl
from jax.experimental.pallas import tpu as pltpu

N = 1024
DIN = 256
H = 8
DH = 32
HDH = H * DH
NEG_SLOPE = 0.2
ALPHA = 0.8
BETA = 0.5

BU = 256
NU = N // BU


def _fused_kernel(feat_ref,
                  adj0_ref, adj1_ref, adj2_ref,
                  W0_ref, alm0_ref, arm0_ref, b0_ref,
                  W1_ref, alm1_ref, arm1_ref, b1_ref,
                  W2_ref, alm2_ref, arm2_ref, b2_ref,
                  qW_ref, qb_ref, kW_ref, kb_ref, mvW_ref, mvb_ref,
                  mv_ref, res_ref,
                  h_s, A1_s, A2_s, B1_s, B2_s, num_s, den_s):
    ui = pl.program_id(0)
    adj_refs = (adj0_ref, adj1_ref, adj2_ref)
    W_refs = (W0_ref, W1_ref, W2_ref)
    alm_refs = (alm0_ref, alm1_ref, alm2_ref)
    arm_refs = (arm0_ref, arm1_ref, arm2_ref)
    b_refs = (b0_ref, b1_ref, b2_ref)

    @pl.when(ui == 0)
    def _setup():
        feat = feat_ref[...]
        for v in range(3):
            h = jnp.dot(feat, W_refs[v][...], preferred_element_type=jnp.float32)
            h_s[v] = h
            el = jnp.dot(h, alm_refs[v][...], preferred_element_type=jnp.float32)
            dn = (((0,), (1,)), ((), ()))
            erT = lax.dot_general(arm_refs[v][...], h, dn,
                                  preferred_element_type=jnp.float32)
            elT = lax.dot_general(alm_refs[v][...], h, dn,
                                  preferred_element_type=jnp.float32)
            elmax_col = jnp.max(elT, axis=1, keepdims=True)
            elmax_row = jnp.max(el, axis=0, keepdims=True)
            m = elmax_col + erT
            MT = jnp.where(m > 0, m, NEG_SLOPE * m)
            A1_s[v] = jnp.exp(el - elmax_row)
            A2_s[v] = jnp.exp(NEG_SLOPE * (el - elmax_row))
            B1_s[v] = jnp.exp(erT + elmax_col - MT)
            B2_s[v] = jnp.exp(NEG_SLOPE * (erT + elmax_col) - MT)
        num_s[...] = jnp.zeros_like(num_s)
        den_s[...] = jnp.zeros_like(den_s)

    rows = ui * BU + lax.broadcasted_iota(jnp.int32, (BU, N), 0)
    cols = lax.broadcasted_iota(jnp.int32, (BU, N), 1)
    diag = rows == cols
    ones_col = jnp.ones((BU, 1), dtype=jnp.float32)
    dn0 = (((0,), (0,)), ((), ()))

    for v in range(3):
        maskf = jnp.where((adj_refs[v][...] > 0.0) | diag, 1.0, 0.0)
        for hh in range(H):
            a1 = A1_s[v, pl.ds(ui * BU, BU), hh:hh + 1]
            a2 = A2_s[v, pl.ds(ui * BU, BU), hh:hh + 1]
            b1 = B1_s[v, hh:hh + 1, :]
            b2 = B2_s[v, hh:hh + 1, :]
            p = jnp.maximum(a1 * b1, a2 * b2) * maskf
            hsl = h_s[v, pl.ds(ui * BU, BU), hh * DH:(hh + 1) * DH]
            num_s[v, :, hh * DH:(hh + 1) * DH] += lax.dot_general(
                p, hsl, dn0, preferred_element_type=jnp.float32)
            den_s[v, :, hh:hh + 1] += lax.dot_general(
                p, ones_col, dn0, preferred_element_type=jnp.float32)

    @pl.when(ui == NU - 1)
    def _finish():
        views = []
        for v in range(3):
            cols_out = []
            for hh in range(H):
                sl = slice(hh * DH, (hh + 1) * DH)
                o = (num_s[v, :, sl] / den_s[v, :, hh:hh + 1]
                     + b_refs[v][0, sl])
                cols_out.append(jnp.maximum(o, 0.0))
            views.append(jnp.concatenate(cols_out, axis=1))

        d_k = jnp.sqrt(jnp.float32(DH * N))
        qW = qW_ref[...]
        kW = kW_ref[...]
        mvW = mvW_ref[...]
        logits = []
        gs = []
        for v in range(3):
            Qv = jnp.dot(views[v], qW, preferred_element_type=jnp.float32) + qb_ref[0]
            Kv = jnp.dot(views[v], kW, preferred_element_type=jnp.float32) + kb_ref[0]
            logits.append(jnp.sum(Qv * Kv) / d_k)
            gs.append(jnp.sum(views[v] * mvW))
        m = jnp.maximum(jnp.maximum(logits[0], logits[1]), logits[2])
        ex = [jnp.exp(l - m) for l in logits]
        tot = ex[0] + ex[1] + ex[2]
        mvb = mvb_ref[0, 0]
        c = [ALPHA * (e / tot) + (1.0 - ALPHA) for e in ex]
        omega = [jax.nn.sigmoid(c[v] * gs[v] + mvb) for v in range(3)]
        mv = (omega[0] * c[0] * views[0] + omega[1] * c[1] * views[1]
              + omega[2] * c[2] * views[2])
        mv_ref[...] = mv
        for v in range(3):
            res_ref[v] = BETA * c[v] * views[v] + (1.0 - BETA) * mv


def _block_diag_attn(a):
    out = jnp.zeros((H, DH, H), dtype=a.dtype)
    out = out.at[jnp.arange(H), :, jnp.arange(H)].set(a)
    return out.reshape(HDH, H)


@jax.jit
def kernel(feature, s_adj, t_adj, poi_adj,
           sW, s_al, s_ar, s_b,
           tW, t_al, t_ar, t_b,
           pW, p_al, p_ar, p_b,
           qW, qb, kW, kb, mvW, mvb):
    full = lambda *shape: pl.BlockSpec(shape, lambda ui: (0,) * len(shape))
    per_view_specs = []
    for _ in range(3):
        per_view_specs += [
            full(DIN, HDH),
            full(HDH, H),
            full(HDH, H),
            full(1, HDH),
        ]

    mv_out, result = pl.pallas_call(
        _fused_kernel,
        grid=(NU,),
        in_specs=[
            full(N, DIN),
            pl.BlockSpec((BU, N), lambda ui: (ui, 0)),
            pl.BlockSpec((BU, N), lambda ui: (ui, 0)),
            pl.BlockSpec((BU, N), lambda ui: (ui, 0)),
            *per_view_specs,
            full(HDH, DH),
            full(1, DH),
            full(HDH, DH),
            full(1, DH),
            full(N, HDH),
            full(1, 1),
        ],
        out_specs=[
            full(N, HDH),
            pl.BlockSpec((3, N, HDH), lambda ui: (0, 0, 0)),
        ],
        out_shape=[
            jax.ShapeDtypeStruct((N, HDH), jnp.float32),
            jax.ShapeDtypeStruct((3, N, HDH), jnp.float32),
        ],
        scratch_shapes=[
            pltpu.VMEM((3, N, HDH), jnp.float32),
            pltpu.VMEM((3, N, H), jnp.float32),
            pltpu.VMEM((3, N, H), jnp.float32),
            pltpu.VMEM((3, H, N), jnp.float32),
            pltpu.VMEM((3, H, N), jnp.float32),
            pltpu.VMEM((3, N, HDH), jnp.float32),
            pltpu.VMEM((3, N, H), jnp.float32),
        ],
    )(feature, s_adj, t_adj, poi_adj,
      sW, _block_diag_attn(s_al), _block_diag_attn(s_ar), s_b.reshape(1, HDH),
      tW, _block_diag_attn(t_al), _block_diag_attn(t_ar), t_b.reshape(1, HDH),
      pW, _block_diag_attn(p_al), _block_diag_attn(p_ar), p_b.reshape(1, HDH),
      qW, qb.reshape(1, DH), kW, kb.reshape(1, DH),
      mvW.reshape(N, HDH), mvb.reshape(1, 1))

    return (mv_out, result)
